# Initial kernel scaffold; baseline (speedup 1.0000x reference)
import jax
import jax.numpy as jnp
from jax.experimental import pallas as pl


def kernel(graph, emb):
    raise NotImplementedError("write your pallas kernel here")


# speedup vs baseline: 1.3784x; 1.3784x over previous
import functools

import jax
import jax.numpy as jnp
from jax import lax
from jax.experimental import pallas as pl
from jax.experimental.pallas import tpu as pltpu
from jax.experimental.pallas import tpu_sc as plsc

N_NODES = 100000
HIDDEN = 128
CHUNK = 125
NCHUNK = N_NODES // CHUNK


@functools.partial(jax.jit, static_argnums=())
def _gather_sc(table, idx2d):
    info = plsc.get_sparse_core_info()
    nw = info.num_cores * info.num_subcores
    chunks_per_w = NCHUNK // nw

    mesh = plsc.VectorSubcoreMesh(core_axis_name="c", subcore_axis_name="s")

    @functools.partial(
        pl.kernel,
        mesh=mesh,
        out_type=jax.ShapeDtypeStruct((NCHUNK, CHUNK, HIDDEN), jnp.float32),
        scratch_types=[
            pltpu.VMEM((CHUNK,), jnp.int32),
            pltpu.VMEM((CHUNK, HIDDEN), jnp.float32),
            pltpu.SemaphoreType.DMA,
        ],
    )
    def k(table_hbm, idx_hbm, out_hbm, idx_v, rows_v, sem):
        wid = lax.axis_index("s") * info.num_cores + lax.axis_index("c")
        base = wid * chunks_per_w

        def body(i, _):
            chunk = base + i
            pltpu.sync_copy(idx_hbm.at[chunk], idx_v)
            pltpu.async_copy(table_hbm.at[idx_v], rows_v, sem).wait()
            pltpu.sync_copy(rows_v, out_hbm.at[chunk])
            return ()

        lax.fori_loop(0, chunks_per_w, body, ())

    return k(table, idx2d)


def kernel(graph, emb):
    table = emb[0]
    idx2d = graph.reshape(NCHUNK, CHUNK).astype(jnp.int32)
    out = _gather_sc(table, idx2d)
    return out.reshape(N_NODES, HIDDEN)


# speedup vs baseline: 1.4077x; 1.0213x over previous
import functools

import jax
import jax.numpy as jnp
from jax import lax
from jax.experimental import pallas as pl
from jax.experimental.pallas import tpu as pltpu
from jax.experimental.pallas import tpu_sc as plsc

N_NODES = 100000
HIDDEN = 128
CHUNK = 125
NCHUNK = N_NODES // CHUNK
NBUF = 5
NW = 32


@functools.partial(jax.jit, static_argnums=())
def _gather_sc(table, idx2d):
    info = plsc.get_sparse_core_info()
    chunks_per_w = NCHUNK // NW
    n_outer = chunks_per_w // NBUF

    mesh = plsc.VectorSubcoreMesh(core_axis_name="c", subcore_axis_name="s")

    @functools.partial(
        pl.kernel,
        mesh=mesh,
        out_type=jax.ShapeDtypeStruct((NCHUNK, CHUNK, HIDDEN), jnp.float32),
        scratch_types=[
            pltpu.VMEM((chunks_per_w, CHUNK), jnp.int32),
            pltpu.VMEM((NBUF, CHUNK, HIDDEN), jnp.float32),
        ] + [pltpu.SemaphoreType.DMA] * (2 * NBUF),
    )
    def k(table_hbm, idx_hbm, out_hbm, idx_v, rows_v, *sems):
        gsems, ssems = sems[:NBUF], sems[NBUF:]
        wid = lax.axis_index("s") * info.num_cores + lax.axis_index("c")
        base = wid * chunks_per_w

        def gather(i, j):
            return pltpu.make_async_copy(
                table_hbm.at[idx_v.at[i]], rows_v.at[j], gsems[j])

        def store(i, j):
            return pltpu.make_async_copy(
                rows_v.at[j], out_hbm.at[base + i], ssems[j])

        pltpu.sync_copy(idx_hbm.at[wid], idx_v)
        for j in range(NBUF):
            gather(j, j).start()

        def body(o, _):
            for j in range(NBUF):
                i = o * NBUF + j
                gather(i, j).wait()
                store(i, j).start()

                @pl.when(o < n_outer - 1)
                def _():
                    store(i, j).wait()
                    gather(i + NBUF, j).start()
            return ()

        lax.fori_loop(0, n_outer, body, ())
        for j in range(NBUF):
            store((n_outer - 1) * NBUF + j, j).wait()

    return k(table, idx2d)


def kernel(graph, emb):
    table = emb[0]
    idx3d = graph.reshape(NW, NCHUNK // NW, CHUNK).astype(jnp.int32)
    out = _gather_sc(table, idx3d)
    return out.reshape(N_NODES, HIDDEN)


# speedup vs baseline: 2.8909x; 2.0536x over previous
import functools

import jax
import jax.numpy as jnp
from jax import lax
from jax.experimental import pallas as pl
from jax.experimental.pallas import tpu as pltpu
from jax.experimental.pallas import tpu_sc as plsc

N_NODES = 100000
HIDDEN = 128
CHUNK = 125
NCHUNK = N_NODES // CHUNK
NBUF = 5
NW = 32


@functools.partial(jax.jit, static_argnums=())
def _gather_sc(table, idx2d):
    info = plsc.get_sparse_core_info()
    chunks_per_w = NCHUNK // NW
    n_outer = chunks_per_w // NBUF

    mesh = plsc.VectorSubcoreMesh(core_axis_name="c", subcore_axis_name="s")

    @functools.partial(
        pl.kernel,
        mesh=mesh,
        out_type=jax.ShapeDtypeStruct((NCHUNK, CHUNK, HIDDEN), jnp.float32),
        scratch_types=[
            pltpu.VMEM((chunks_per_w, CHUNK), jnp.int32),
            pltpu.VMEM((NBUF, CHUNK, HIDDEN), jnp.float32),
            pltpu.VMEM_SHARED((100, HIDDEN), jnp.float32),
        ] + [pltpu.SemaphoreType.DMA] * (2 * NBUF),
    )
    def k(table_hbm, idx_hbm, out_hbm, idx_v, rows_v, table_sh, *sems):
        gsems, ssems = sems[:NBUF], sems[NBUF:]
        sid = lax.axis_index("s")
        wid = sid * info.num_cores + lax.axis_index("c")
        base = wid * chunks_per_w

        @pl.when(sid == 0)
        def _():
            pltpu.sync_copy(table_hbm, table_sh)
        plsc.subcore_barrier()

        def gather(i, j):
            return pltpu.make_async_copy(
                table_sh.at[idx_v.at[i]], rows_v.at[j], gsems[j])

        def store(i, j):
            return pltpu.make_async_copy(
                rows_v.at[j], out_hbm.at[base + i], ssems[j])

        pltpu.sync_copy(idx_hbm.at[wid], idx_v)
        for j in range(NBUF):
            gather(j, j).start()

        def body(o, _):
            for j in range(NBUF):
                i = o * NBUF + j
                gather(i, j).wait()
                store(i, j).start()

                @pl.when(o < n_outer - 1)
                def _():
                    store(i, j).wait()
                    gather(i + NBUF, j).start()
            return ()

        lax.fori_loop(0, n_outer, body, ())
        for j in range(NBUF):
            store((n_outer - 1) * NBUF + j, j).wait()

    return k(table, idx2d)


def kernel(graph, emb):
    table = emb[0]
    idx3d = graph.reshape(NW, NCHUNK // NW, CHUNK).astype(jnp.int32)
    out = _gather_sc(table, idx3d)
    return out.reshape(N_NODES, HIDDEN)
